# baseline (speedup 1.0000x reference)
import functools

import jax
import jax.numpy as jnp
from jax import lax
from jax.experimental import pallas as pl
from jax.experimental.pallas import tpu as pltpu
from jax.experimental.pallas import tpu_sc as plsc

N = 10000
E = 320000
R = 8
D_IN = 128
H = 32
VGAE = 32
HALF = N // 2

NB = 2000
NC = 2
NS = 16
NW = NC * NS
CH = 128
CHUNKS = 80
MCH = 10
MEGA = MCH * CH
MEGS = CHUNKS // MCH
NBUF = 2
EPAD = NW * CHUNKS * CH
NPAD = 10240
ROWS_PER_TILE = NPAD // NS


def _proj_body(x_ref, w_ref, loop_ref, b_ref, proj_ref, self_ref):
    xb = x_ref[...]
    for r in range(R):
        proj_ref[r] = jnp.dot(xb, w_ref[r], preferred_element_type=jnp.float32)
    self_ref[...] = (
        jnp.dot(xb, loop_ref[...], preferred_element_type=jnp.float32)
        + b_ref[...] + proj_ref[0]
    )


def _project(xl, W, loop_w, b):
    D = xl.shape[1]
    return pl.pallas_call(
        _proj_body,
        grid=(N // NB,),
        in_specs=[
            pl.BlockSpec((NB, D), lambda i: (i, 0)),
            pl.BlockSpec((R, D, H), lambda i: (0, 0, 0)),
            pl.BlockSpec((D, H), lambda i: (0, 0)),
            pl.BlockSpec((H,), lambda i: (0,)),
        ],
        out_specs=[
            pl.BlockSpec((R, NB, H), lambda i: (0, i, 0)),
            pl.BlockSpec((NB, H), lambda i: (i, 0)),
        ],
        out_shape=[
            jax.ShapeDtypeStruct((R, N, H), jnp.float32),
            jax.ShapeDtypeStruct((N, H), jnp.float32),
        ],
    )(xl, W, loop_w, b)


def _sc_body(table_hbm, gidx_hbm, dst_hbm, zeros_hbm, out_hbm,
             gidx_v, dst_v, rows_v, acc_sh, *sems):
    cid = lax.axis_index("c")
    sid = lax.axis_index("s")
    pltpu.sync_copy(gidx_hbm.at[cid, sid], gidx_v)
    pltpu.sync_copy(dst_hbm.at[cid, sid], dst_v)
    base = sid * ROWS_PER_TILE
    pltpu.sync_copy(zeros_hbm.at[pl.ds(base, ROWS_PER_TILE)],
                    acc_sh.at[pl.ds(base, ROWS_PER_TILE)])
    plsc.subcore_barrier()

    def gather(jj, b):
        pltpu.make_async_copy(table_hbm.at[gidx_v.at[pl.ds(jj * MEGA, MEGA)]],
                              rows_v.at[b], sems[b]).start()

    def drain_scatter(jj, b):
        pltpu.make_async_copy(table_hbm.at[gidx_v.at[pl.ds(jj * MEGA, MEGA)]],
                              rows_v.at[b], sems[b]).wait()
        for k in range(MCH):
            pltpu.sync_copy(rows_v.at[b].at[pl.ds(k * CH, CH)],
                            acc_sh.at[dst_v.at[jj * MCH + k]], add=True)

    for b in range(NBUF):
        gather(b, b)

    def body(i, carry):
        j = NBUF * i
        for b in range(NBUF):
            jj = j + b
            drain_scatter(jj, b)

            @pl.when(jj + NBUF < MEGS)
            def _(jj=jj, b=b):
                gather(jj + NBUF, b)
        return carry

    lax.fori_loop(0, MEGS // NBUF, body, 0)
    plsc.subcore_barrier()
    pltpu.sync_copy(acc_sh.at[pl.ds(base, ROWS_PER_TILE)],
                    out_hbm.at[cid, pl.ds(base, ROWS_PER_TILE)])


_sc_edge_sum = functools.partial(
    pl.kernel,
    mesh=plsc.VectorSubcoreMesh(core_axis_name="c", subcore_axis_name="s"),
    compiler_params=pltpu.CompilerParams(use_tc_tiling_on_sc=False),
    out_type=jax.ShapeDtypeStruct((NC, NPAD, H), jnp.float32),
    scratch_types=[
        pltpu.VMEM((CHUNKS * CH,), jnp.int32),
        pltpu.VMEM((CHUNKS, CH), jnp.int32),
        pltpu.VMEM((NBUF, MEGA, H), jnp.float32),
        pltpu.VMEM_SHARED((NPAD, H), jnp.float32),
    ] + [pltpu.SemaphoreType.DMA] * NBUF,
)(_sc_body)


def _combine_body(p_ref, s_ref, o_ref):
    o_ref[...] = jnp.tanh(p_ref[0] + p_ref[1] + s_ref[...])


def _combine(partial, selft):
    return pl.pallas_call(
        _combine_body,
        grid=(N // NB,),
        in_specs=[
            pl.BlockSpec((NC, NB, H), lambda i: (0, i, 0)),
            pl.BlockSpec((NB, H), lambda i: (i, 0)),
        ],
        out_specs=pl.BlockSpec((NB, H), lambda i: (i, 0)),
        out_shape=jax.ShapeDtypeStruct((N, H), jnp.float32),
    )(partial, selft)


def _head_body(h0, h1, h2, nz, wmu, bmu, wstd, bstd, w1, b1, w2, b2, out_ref):
    a0, a1, a2 = h0[...], h1[...], h2[...]

    def lin3(w, bias):
        return (jnp.dot(a0, w[0:H], preferred_element_type=jnp.float32)
                + jnp.dot(a1, w[H:2 * H], preferred_element_type=jnp.float32)
                + jnp.dot(a2, w[2 * H:3 * H], preferred_element_type=jnp.float32)
                + bias[...])

    mean = lin3(wmu, bmu)
    log_std = lin3(wstd, bstd)
    z = mean + nz[...] * jnp.exp(log_std)
    zh = jnp.concatenate([z[:HALF], z[HALF:]], axis=1)
    hh = jnp.maximum(jnp.dot(zh, w1[...], preferred_element_type=jnp.float32)
                     + b1[...], 0.0)
    o = jnp.dot(hh, w2[...], preferred_element_type=jnp.float32) + b2[...]
    out_ref[...] = 1.0 / (1.0 + jnp.exp(-o))


def _head(h0, h1, h2, noise, wmu, bmu, wstd, bstd, w1, b1, w2, b2):
    return pl.pallas_call(
        _head_body,
        out_shape=jax.ShapeDtypeStruct((HALF, 1), jnp.float32),
    )(h0, h1, h2, noise, wmu, bmu, wstd, bstd, w1, b1, w2, b2)


def kernel(x, edge_index, etypes, nlabel,
           W0, b0, loop0, W1, b1, loop1, W2, b2, loop2,
           Wmu, bmu, Wstd, bstd, lin1_w, lin1_b, lin2_w, lin2_b):
    src = edge_index[0].astype(jnp.int32)
    dst = edge_index[1].astype(jnp.int32)
    et = etypes.astype(jnp.int32)
    gidx = et * N + src
    pad = EPAD - E
    gidx_p = jnp.concatenate(
        [gidx, jnp.zeros((pad,), jnp.int32)]).reshape(NC, NS, CHUNKS * CH)
    dst_p = jnp.concatenate(
        [dst, jnp.full((pad,), NPAD - 1, jnp.int32)]).reshape(NC, NS, CHUNKS, CH)
    zeros = jnp.zeros((NPAD, H), jnp.float32)

    xl = x
    hs = []
    for W, b, lw in ((W0, b0, loop0), (W1, b1, loop1), (W2, b2, loop2)):
        proj, selft = _project(xl, W, lw, b)
        table = proj.reshape(R * N, H)
        partial = _sc_edge_sum(table, gidx_p, dst_p, zeros)
        xl = _combine(partial, selft)
        hs.append(xl)

    noise = jax.random.normal(jax.random.key(42), (N, VGAE), jnp.float32)
    out2d = _head(hs[0], hs[1], hs[2], noise,
                  Wmu, bmu, Wstd, bstd, lin1_w, lin1_b, lin2_w, lin2_b)
    return out2d[:, 0]


# speedup vs baseline: 1.2102x; 1.2102x over previous
import functools

import jax
import jax.numpy as jnp
from jax import lax
from jax.experimental import pallas as pl
from jax.experimental.pallas import tpu as pltpu
from jax.experimental.pallas import tpu_sc as plsc

N = 10000
E = 320000
R = 8
D_IN = 128
H = 32
VGAE = 32
HALF = N // 2

NC = 2
NS = 16
NW = NC * NS
CH = 128
CHUNKS = 80
MCH = 10
MEGA = MCH * CH
MEGS = CHUNKS // MCH
NBUF = 2
EPAD = NW * CHUNKS * CH
NPAD = 10240
ROWS_PER_TILE = NPAD // NS
NP4 = NPAD // 4
NG4 = 5
NB4 = NP4 // NG4


def _kron4(w):
    d, o = w.shape
    return jnp.einsum('qp,dj->qdpj', jnp.eye(4, dtype=w.dtype),
                      w).reshape(4 * d, 4 * o)


def _proj_body(x_ref, wq_ref, ws_ref, b_ref, t_ref, s_ref):
    xb = x_ref[...]
    t_ref[...] = jnp.dot(xb, wq_ref[0], preferred_element_type=jnp.float32)

    @pl.when(pl.program_id(1) == 0)
    def _():
        s_ref[...] = (jnp.dot(xb, ws_ref[...],
                              preferred_element_type=jnp.float32) + b_ref[...])


def _project(x4, wq, wsq, b4):
    d4 = x4.shape[1]
    return pl.pallas_call(
        _proj_body,
        grid=(NG4, R),
        in_specs=[
            pl.BlockSpec((NB4, d4), lambda i, r: (i, 0)),
            pl.BlockSpec((1, d4, 128), lambda i, r: (r, 0, 0)),
            pl.BlockSpec((d4, 128), lambda i, r: (0, 0)),
            pl.BlockSpec((128,), lambda i, r: (0,)),
        ],
        out_specs=[
            pl.BlockSpec((NB4, 128), lambda i, r: (NG4 * r + i, 0)),
            pl.BlockSpec((NB4, 128), lambda i, r: (i, 0)),
        ],
        out_shape=[
            jax.ShapeDtypeStruct((R * NP4, 128), jnp.float32),
            jax.ShapeDtypeStruct((NP4, 128), jnp.float32),
        ],
    )(x4, wq, wsq, b4)


def _sc_body(table_hbm, gidx_hbm, dst_hbm, zeros_hbm, out_hbm,
             gidx_v, dst_v, rows_v, acc_sh, *sems):
    cid = lax.axis_index("c")
    sid = lax.axis_index("s")
    pltpu.sync_copy(gidx_hbm.at[cid, sid], gidx_v)
    pltpu.sync_copy(dst_hbm.at[cid, sid], dst_v)
    base = sid * ROWS_PER_TILE
    pltpu.sync_copy(zeros_hbm.at[pl.ds(base, ROWS_PER_TILE)],
                    acc_sh.at[pl.ds(base, ROWS_PER_TILE)])
    plsc.subcore_barrier()

    def gather(jj, b):
        pltpu.make_async_copy(table_hbm.at[gidx_v.at[pl.ds(jj * MEGA, MEGA)]],
                              rows_v.at[b], sems[b]).start()

    def drain_scatter(jj, b):
        pltpu.make_async_copy(table_hbm.at[gidx_v.at[pl.ds(jj * MEGA, MEGA)]],
                              rows_v.at[b], sems[b]).wait()
        for k in range(MCH):
            pltpu.sync_copy(rows_v.at[b].at[pl.ds(k * CH, CH)],
                            acc_sh.at[dst_v.at[jj * MCH + k]], add=True)

    for b in range(NBUF):
        gather(b, b)

    def body(i, carry):
        j = NBUF * i
        for b in range(NBUF):
            jj = j + b
            drain_scatter(jj, b)

            @pl.when(jj + NBUF < MEGS)
            def _(jj=jj, b=b):
                gather(jj + NBUF, b)
        return carry

    lax.fori_loop(0, MEGS // NBUF, body, 0)
    plsc.subcore_barrier()
    pltpu.sync_copy(acc_sh.at[pl.ds(base, ROWS_PER_TILE)],
                    out_hbm.at[cid, pl.ds(base, ROWS_PER_TILE)])


_sc_edge_sum = functools.partial(
    pl.kernel,
    mesh=plsc.VectorSubcoreMesh(core_axis_name="c", subcore_axis_name="s"),
    compiler_params=pltpu.CompilerParams(use_tc_tiling_on_sc=False),
    out_type=jax.ShapeDtypeStruct((NC, NPAD, H), jnp.float32),
    scratch_types=[
        pltpu.VMEM((CHUNKS * CH,), jnp.int32),
        pltpu.VMEM((CHUNKS, CH), jnp.int32),
        pltpu.VMEM((NBUF, MEGA, H), jnp.float32),
        pltpu.VMEM_SHARED((NPAD, H), jnp.float32),
    ] + [pltpu.SemaphoreType.DMA] * NBUF,
)(_sc_body)


def _combine_body(p_ref, s_ref, o_ref):
    o_ref[...] = jnp.tanh(p_ref[0] + p_ref[1] + s_ref[...])


def _combine(partial4, s4):
    return pl.pallas_call(
        _combine_body,
        grid=(NG4,),
        in_specs=[
            pl.BlockSpec((NC, NB4, 128), lambda i: (0, i, 0)),
            pl.BlockSpec((NB4, 128), lambda i: (i, 0)),
        ],
        out_specs=pl.BlockSpec((NB4, 128), lambda i: (i, 0)),
        out_shape=jax.ShapeDtypeStruct((NP4, 128), jnp.float32),
    )(partial4, s4)


def _head_body(h0, h1, h2, nz, kmu, bmu4, kstd, bstd4, k1, b14, k2, b24,
               out_ref):
    def lin3(kw, bias):
        acc = bias[...]
        for hl, l in ((h0, 0), (h1, 1), (h2, 2)):
            acc = acc + jnp.dot(hl[...], kw[l],
                                preferred_element_type=jnp.float32)
        return acc

    mean = lin3(kmu, bmu4)
    log_std = lin3(kstd, bstd4)
    z = mean + nz[...] * jnp.exp(log_std)
    zh = jnp.concatenate([z[:HALF // 4], z[HALF // 4:2 * (HALF // 4)]], axis=1)
    hh = jnp.maximum(jnp.dot(zh, k1[...], preferred_element_type=jnp.float32)
                     + b14[...], 0.0)
    o = jnp.dot(hh, k2[...], preferred_element_type=jnp.float32) + b24[...]
    out_ref[...] = 1.0 / (1.0 + jnp.exp(-o))


def _head(h0, h1, h2, noise4, kmu, bmu4, kstd, bstd4, k1, b14, k2, b24):
    return pl.pallas_call(
        _head_body,
        out_shape=jax.ShapeDtypeStruct((HALF // 4, 4), jnp.float32),
    )(h0, h1, h2, noise4, kmu, bmu4, kstd, bstd4, k1, b14, k2, b24)


def kernel(x, edge_index, etypes, nlabel,
           W0, b0, loop0, W1, b1, loop1, W2, b2, loop2,
           Wmu, bmu, Wstd, bstd, lin1_w, lin1_b, lin2_w, lin2_b):
    src = edge_index[0].astype(jnp.int32)
    dst = edge_index[1].astype(jnp.int32)
    et = etypes.astype(jnp.int32)
    gidx = et * NPAD + src
    pad = EPAD - E
    gidx_p = jnp.concatenate(
        [gidx, jnp.zeros((pad,), jnp.int32)]).reshape(NC, NS, CHUNKS * CH)
    dst_p = jnp.concatenate(
        [dst, jnp.full((pad,), NPAD - 1, jnp.int32)]).reshape(NC, NS, CHUNKS, CH)
    zeros = jnp.zeros((NPAD, H), jnp.float32)

    x4 = jnp.concatenate(
        [x, jnp.zeros((NPAD - N, D_IN), jnp.float32)]).reshape(NP4, 4 * D_IN)
    hs = []
    for W, b, lw in ((W0, b0, loop0), (W1, b1, loop1), (W2, b2, loop2)):
        wq = jax.vmap(_kron4)(W).reshape(R, -1, 128)
        wsq = _kron4(lw + W[0])
        b4 = jnp.tile(b, 4)
        table4, s4 = _project(x4, wq, wsq, b4)
        partial = _sc_edge_sum(table4.reshape(R * NPAD, H), gidx_p, dst_p,
                               zeros)
        x4 = _combine(partial.reshape(NC, NP4, 128), s4)
        hs.append(x4)

    noise = jax.random.normal(jax.random.key(42), (N, VGAE), jnp.float32)
    noise4 = jnp.concatenate(
        [noise, jnp.zeros((NPAD - N, VGAE), jnp.float32)]).reshape(NP4, 128)
    kmu = jnp.stack([_kron4(Wmu[l * H:(l + 1) * H]) for l in range(3)])
    kstd = jnp.stack([_kron4(Wstd[l * H:(l + 1) * H]) for l in range(3)])
    k1 = jnp.concatenate([_kron4(lin1_w[:VGAE]), _kron4(lin1_w[VGAE:])], axis=0)
    out4 = _head(hs[0], hs[1], hs[2], noise4,
                 kmu, jnp.tile(bmu, 4), kstd, jnp.tile(bstd, 4),
                 k1, jnp.tile(lin1_b, 4),
                 _kron4(lin2_w), jnp.tile(lin2_b, 4))
    return out4.reshape(HALF)


# speedup vs baseline: 1.2126x; 1.0020x over previous
import functools

import jax
import jax.numpy as jnp
from jax import lax
from jax.experimental import pallas as pl
from jax.experimental.pallas import tpu as pltpu
from jax.experimental.pallas import tpu_sc as plsc

N = 10000
E = 320000
R = 8
D_IN = 128
H = 32
VGAE = 32
HALF = N // 2

NC = 2
NS = 16
NW = NC * NS
CH = 128
CF = 128
CS = 32
MCH = 8
MEGA = MCH * CH
NBUF = 2
EPAD = NS * (CF + CS) * CH
EXTRA = (CF - CS) * CH
NPAD = 10240
ROWS_PER_TILE = NPAD // NS
NP4 = NPAD // 4
NG4 = 5
NB4 = NP4 // NG4


def _kron4(w):
    d, o = w.shape
    return jnp.einsum('qp,dj->qdpj', jnp.eye(4, dtype=w.dtype),
                      w).reshape(4 * d, 4 * o)


def _proj_body(x_ref, wq_ref, ws_ref, b_ref, t_ref, s_ref):
    xb = x_ref[...]
    t_ref[...] = jnp.dot(xb, wq_ref[0], preferred_element_type=jnp.float32)

    @pl.when(pl.program_id(1) == 0)
    def _():
        s_ref[...] = (jnp.dot(xb, ws_ref[...],
                              preferred_element_type=jnp.float32) + b_ref[...])


def _project(x4, wq, wsq, b4):
    d4 = x4.shape[1]
    return pl.pallas_call(
        _proj_body,
        grid=(NG4, R),
        in_specs=[
            pl.BlockSpec((NB4, d4), lambda i, r: (i, 0)),
            pl.BlockSpec((1, d4, 128), lambda i, r: (r, 0, 0)),
            pl.BlockSpec((d4, 128), lambda i, r: (0, 0)),
            pl.BlockSpec((128,), lambda i, r: (0,)),
        ],
        out_specs=[
            pl.BlockSpec((NB4, 128), lambda i, r: (NG4 * r + i, 0)),
            pl.BlockSpec((NB4, 128), lambda i, r: (i, 0)),
        ],
        out_shape=[
            jax.ShapeDtypeStruct((R * NP4, 128), jnp.float32),
            jax.ShapeDtypeStruct((NP4, 128), jnp.float32),
        ],
    )(x4, wq, wsq, b4)


def _sc_body(table_hbm, gidx_hbm, dst_hbm, zeros_hbm, out_hbm,
             gidx_v, dst_v, rows_v, acc_sh, *sems):
    cid = lax.axis_index("c")
    sid = lax.axis_index("s")
    nmegs = jnp.where(cid == 0, CF // MCH, CS // MCH)
    pltpu.sync_copy(gidx_hbm.at[cid, sid], gidx_v)
    pltpu.sync_copy(dst_hbm.at[cid, sid], dst_v)
    base = sid * ROWS_PER_TILE
    pltpu.sync_copy(zeros_hbm.at[pl.ds(base, ROWS_PER_TILE)],
                    acc_sh.at[pl.ds(base, ROWS_PER_TILE)])
    plsc.subcore_barrier()

    def gather(jj, b):
        pltpu.make_async_copy(table_hbm.at[gidx_v.at[pl.ds(jj * MEGA, MEGA)]],
                              rows_v.at[b], sems[b]).start()

    def drain_scatter(jj, b):
        pltpu.make_async_copy(table_hbm.at[gidx_v.at[pl.ds(jj * MEGA, MEGA)]],
                              rows_v.at[b], sems[b]).wait()
        for k in range(MCH):
            pltpu.sync_copy(rows_v.at[b].at[pl.ds(k * CH, CH)],
                            acc_sh.at[dst_v.at[jj * MCH + k]], add=True)

    for b in range(NBUF):
        gather(b, b)

    def body(i, carry):
        j = NBUF * i
        for b in range(NBUF):
            jj = j + b
            drain_scatter(jj, b)

            @pl.when(jj + NBUF < nmegs)
            def _(jj=jj, b=b):
                gather(jj + NBUF, b)
        return carry

    lax.fori_loop(0, nmegs // NBUF, body, 0)
    plsc.subcore_barrier()
    pltpu.sync_copy(acc_sh.at[pl.ds(base, ROWS_PER_TILE)],
                    out_hbm.at[cid, pl.ds(base, ROWS_PER_TILE)])


_sc_edge_sum = functools.partial(
    pl.kernel,
    mesh=plsc.VectorSubcoreMesh(core_axis_name="c", subcore_axis_name="s"),
    compiler_params=pltpu.CompilerParams(use_tc_tiling_on_sc=False),
    out_type=jax.ShapeDtypeStruct((NC, NPAD, H), jnp.float32),
    scratch_types=[
        pltpu.VMEM((CF * CH,), jnp.int32),
        pltpu.VMEM((CF, CH), jnp.int32),
        pltpu.VMEM((NBUF, MEGA, H), jnp.float32),
        pltpu.VMEM_SHARED((NPAD, H), jnp.float32),
    ] + [pltpu.SemaphoreType.DMA] * NBUF,
)(_sc_body)


def _combine_body(p_ref, s_ref, o_ref):
    o_ref[...] = jnp.tanh(p_ref[0] + p_ref[1] + s_ref[...])


def _combine(partial4, s4):
    return pl.pallas_call(
        _combine_body,
        grid=(NG4,),
        in_specs=[
            pl.BlockSpec((NC, NB4, 128), lambda i: (0, i, 0)),
            pl.BlockSpec((NB4, 128), lambda i: (i, 0)),
        ],
        out_specs=pl.BlockSpec((NB4, 128), lambda i: (i, 0)),
        out_shape=jax.ShapeDtypeStruct((NP4, 128), jnp.float32),
    )(partial4, s4)


def _head_body(h0, h1, h2, nz, kmu, bmu4, kstd, bstd4, k1, b14, k2, b24,
               out_ref):
    def lin3(kw, bias):
        acc = bias[...]
        for hl, l in ((h0, 0), (h1, 1), (h2, 2)):
            acc = acc + jnp.dot(hl[...], kw[l],
                                preferred_element_type=jnp.float32)
        return acc

    mean = lin3(kmu, bmu4)
    log_std = lin3(kstd, bstd4)
    z = mean + nz[...] * jnp.exp(log_std)
    zh = jnp.concatenate([z[:HALF // 4], z[HALF // 4:2 * (HALF // 4)]], axis=1)
    hh = jnp.maximum(jnp.dot(zh, k1[...], preferred_element_type=jnp.float32)
                     + b14[...], 0.0)
    o = jnp.dot(hh, k2[...], preferred_element_type=jnp.float32) + b24[...]
    out_ref[...] = 1.0 / (1.0 + jnp.exp(-o))


def _head(h0, h1, h2, noise4, kmu, bmu4, kstd, bstd4, k1, b14, k2, b24):
    return pl.pallas_call(
        _head_body,
        out_shape=jax.ShapeDtypeStruct((HALF // 4, 4), jnp.float32),
    )(h0, h1, h2, noise4, kmu, bmu4, kstd, bstd4, k1, b14, k2, b24)


def kernel(x, edge_index, etypes, nlabel,
           W0, b0, loop0, W1, b1, loop1, W2, b2, loop2,
           Wmu, bmu, Wstd, bstd, lin1_w, lin1_b, lin2_w, lin2_b):
    src = edge_index[0].astype(jnp.int32)
    dst = edge_index[1].astype(jnp.int32)
    et = etypes.astype(jnp.int32)
    gidx = et * NPAD + src

    def split_pack(flat, fill):
        flat = jnp.concatenate(
            [flat, jnp.full((EPAD - E,), fill, flat.dtype)])
        cut = NS * CF * CH
        part0 = flat[:cut].reshape(NS, CF * CH)
        part1 = flat[cut:].reshape(NS, CS * CH)
        part1 = jnp.pad(part1, ((0, 0), (0, (CF - CS) * CH)),
                        constant_values=fill)
        return jnp.stack([part0, part1])

    gidx_p = split_pack(gidx, 0)
    dst_p = split_pack(dst, NPAD - 1).reshape(NC, NS, CF, CH)
    zeros = jnp.zeros((NPAD, H), jnp.float32)

    x4 = jnp.concatenate(
        [x, jnp.zeros((NPAD - N, D_IN), jnp.float32)]).reshape(NP4, 4 * D_IN)
    hs = []
    for W, b, lw in ((W0, b0, loop0), (W1, b1, loop1), (W2, b2, loop2)):
        wq = jax.vmap(_kron4)(W).reshape(R, -1, 128)
        wsq = _kron4(lw + W[0])
        b4 = jnp.tile(b, 4)
        table4, s4 = _project(x4, wq, wsq, b4)
        partial = _sc_edge_sum(table4.reshape(R * NPAD, H), gidx_p, dst_p,
                               zeros)
        x4 = _combine(partial.reshape(NC, NP4, 128), s4)
        hs.append(x4)

    noise = jax.random.normal(jax.random.key(42), (N, VGAE), jnp.float32)
    noise4 = jnp.concatenate(
        [noise, jnp.zeros((NPAD - N, VGAE), jnp.float32)]).reshape(NP4, 128)
    kmu = jnp.stack([_kron4(Wmu[l * H:(l + 1) * H]) for l in range(3)])
    kstd = jnp.stack([_kron4(Wstd[l * H:(l + 1) * H]) for l in range(3)])
    k1 = jnp.concatenate([_kron4(lin1_w[:VGAE]), _kron4(lin1_w[VGAE:])], axis=0)
    out4 = _head(hs[0], hs[1], hs[2], noise4,
                 kmu, jnp.tile(bmu, 4), kstd, jnp.tile(bstd, 4),
                 k1, jnp.tile(lin1_b, 4),
                 _kron4(lin2_w), jnp.tile(lin2_b, 4))
    return out4.reshape(HALF)
